# baseline (speedup 1.0000x reference)
import dataclasses
import functools

import jax
import jax.numpy as jnp
from jax import lax
from jax.experimental import pallas as pl
from jax.experimental.pallas import tpu as pltpu
from jax.experimental.pallas import tpu_sc as plsc

_B, _N, _D = 16384, 7, 256
_C = 64
_NC, _NS, _L = 2, 16, 16
_NW = _NC * _NS
_BPW = _B // _NW
_CH = 128
_DP = 257
_MR = 72


def _table_kernel(rank_ref, suit_ref, card_ref, w_ref, m_ref):
    row = lax.broadcasted_iota(jnp.int32, (_C, 1), 0)
    valid = row < 52
    ranks = row % 13
    suits = row // 13
    oh_r = jnp.where(
        (ranks == lax.broadcasted_iota(jnp.int32, (_C, 16), 1)) & valid,
        1.0, 0.0)
    oh_s = jnp.where(
        (suits == lax.broadcasted_iota(jnp.int32, (_C, 8), 1)) & valid,
        1.0, 0.0)
    t = (
        lax.dot_general(oh_r, rank_ref[...], (((1,), (0,)), ((), ())),
                        preferred_element_type=jnp.float32)
        + lax.dot_general(oh_s, suit_ref[...], (((1,), (0,)), ((), ())),
                          preferred_element_type=jnp.float32)
        + card_ref[...]
    )
    m_ref[...] = lax.dot_general(
        t, w_ref[...], (((1,), (1,)), ((), ())),
        preferred_element_type=jnp.float32)


def _build_table(rank_emb, suit_emb, card_emb, W):
    rank_pad = jnp.zeros((16, _D), jnp.float32).at[:13].set(rank_emb)
    suit_pad = jnp.zeros((8, _D), jnp.float32).at[:4].set(suit_emb)
    card_pad = jnp.zeros((_C, _D), jnp.float32).at[:52].set(card_emb)
    return pl.pallas_call(
        _table_kernel,
        out_shape=jax.ShapeDtypeStruct((_C, _D), jnp.float32),
    )(rank_pad, suit_pad, card_pad, W)


def _sc_body(cards_hbm, m_hbm, out_hbm, m_v, cards_v, out_v):
    c = lax.axis_index("c")
    s = lax.axis_index("s")
    wid = s * _NC + c
    pltpu.sync_copy(m_hbm, m_v)
    pltpu.sync_copy(cards_hbm.at[wid], cards_v)
    lane = lax.iota(jnp.int32, _L)

    @pl.loop(0, _BPW // _CH)
    def _chunk(ch):
        @pl.loop(0, _CH // _L)
        def _grp(g):
            r0 = ch * _CH + g * _L
            cvs = [cards_v[n, pl.ds(r0, _L)] for n in range(_N + 1)]
            row_idx = g * _L + lane

            @plsc.parallel_loop(0, _D, step=2, unroll=4)
            def _col(d):
                for u in range(2):
                    dv = jnp.full((_L,), d + u, jnp.int32)
                    g0 = plsc.load_gather(m_v, [cvs[0], dv])
                    g1 = plsc.load_gather(m_v, [cvs[1], dv])
                    g2 = plsc.load_gather(m_v, [cvs[2], dv])
                    g3 = plsc.load_gather(m_v, [cvs[3], dv])
                    g4 = plsc.load_gather(m_v, [cvs[4], dv])
                    g5 = plsc.load_gather(m_v, [cvs[5], dv])
                    g6 = plsc.load_gather(m_v, [cvs[6], dv])
                    g7 = plsc.load_gather(m_v, [cvs[7], dv])
                    acc = ((g0 + g1) + (g2 + g3)) + ((g4 + g5) + (g6 + g7))
                    acc = jnp.maximum(acc, 0.0)
                    plsc.store_scatter(out_v, [row_idx, dv], acc)

        pltpu.sync_copy(out_v.at[:, pl.ds(0, _D)],
                        out_hbm.at[pl.ds(wid * _BPW + ch * _CH, _CH)])


def _sc_call(cards_sc, m):
    mesh = plsc.VectorSubcoreMesh(core_axis_name="c", subcore_axis_name="s")
    cp = pltpu.CompilerParams()
    if "needs_layout_passes" in pltpu.CompilerParams.__dataclass_fields__:
        cp = dataclasses.replace(cp, needs_layout_passes=False)
    run = pl.kernel(
        _sc_body,
        mesh=mesh,
        compiler_params=cp,
        out_type=jax.ShapeDtypeStruct((_B, _D), jnp.float32),
        scratch_types=[
            pltpu.VMEM((_MR, _DP), jnp.float32),
            pltpu.VMEM((_N + 1, _BPW), jnp.int32),
            pltpu.VMEM((_CH, _DP), jnp.float32),
        ],
    )
    return run(cards_sc, m)


def kernel(cards, rank_emb, suit_emb, card_emb, W, b):
    m = _build_table(rank_emb, suit_emb, card_emb, W)
    m_pad = jnp.zeros((_MR, _DP), jnp.float32)
    m_pad = m_pad.at[:_C, :_D].set(m).at[_C, :_D].set(b)
    cards_t = cards.reshape(_NW, _BPW, _N).transpose(0, 2, 1)
    bias_row = jnp.full((_NW, 1, _BPW), _C, jnp.int32)
    cards_sc = jnp.concatenate([cards_t, bias_row], axis=1)
    return _sc_call(cards_sc, m_pad)


# speedup vs baseline: 1.0558x; 1.0558x over previous
import dataclasses
import functools

import jax
import jax.numpy as jnp
from jax import lax
from jax.experimental import pallas as pl
from jax.experimental.pallas import tpu as pltpu
from jax.experimental.pallas import tpu_sc as plsc

_B, _N, _D = 16384, 7, 256
_C = 64
_NC, _NS, _L = 2, 16, 16
_NW = _NC * _NS
_BPW = _B // _NW
_CH = 128
_DP = 257
_MR = 72


def _table_kernel(rank_ref, suit_ref, card_ref, w_ref, m_ref):
    row = lax.broadcasted_iota(jnp.int32, (_C, 1), 0)
    valid = row < 52
    ranks = row % 13
    suits = row // 13
    oh_r = jnp.where(
        (ranks == lax.broadcasted_iota(jnp.int32, (_C, 16), 1)) & valid,
        1.0, 0.0)
    oh_s = jnp.where(
        (suits == lax.broadcasted_iota(jnp.int32, (_C, 8), 1)) & valid,
        1.0, 0.0)
    t = (
        lax.dot_general(oh_r, rank_ref[...], (((1,), (0,)), ((), ())),
                        preferred_element_type=jnp.float32)
        + lax.dot_general(oh_s, suit_ref[...], (((1,), (0,)), ((), ())),
                          preferred_element_type=jnp.float32)
        + card_ref[...]
    )
    m_ref[...] = lax.dot_general(
        t, w_ref[...], (((1,), (1,)), ((), ())),
        preferred_element_type=jnp.float32)


def _build_table(rank_emb, suit_emb, card_emb, W):
    rank_pad = jnp.zeros((16, _D), jnp.float32).at[:13].set(rank_emb)
    suit_pad = jnp.zeros((8, _D), jnp.float32).at[:4].set(suit_emb)
    card_pad = jnp.zeros((_C, _D), jnp.float32).at[:52].set(card_emb)
    return pl.pallas_call(
        _table_kernel,
        out_shape=jax.ShapeDtypeStruct((_C, _D), jnp.float32),
    )(rank_pad, suit_pad, card_pad, W)


def _sc_body(cards_hbm, m_hbm, out_hbm, m_v, cards_v, out_v):
    c = lax.axis_index("c")
    s = lax.axis_index("s")
    wid = s * _NC + c
    pltpu.sync_copy(m_hbm, m_v)
    pltpu.sync_copy(cards_hbm.at[wid], cards_v)
    lane = lax.iota(jnp.int32, _L)

    @pl.loop(0, _BPW // _CH)
    def _chunk(ch):
        @pl.loop(0, _CH // _L)
        def _grp(g):
            r0 = ch * _CH + g * _L
            pre = [cards_v[n, pl.ds(r0, _L)] * _D for n in range(_N + 1)]
            srow = (g * _L + lane) * _D

            @plsc.parallel_loop(0, _D, unroll=4,
                                carry=jnp.zeros((_L,), jnp.int32))
            def _col(d, dv):
                g0 = plsc.load_gather(m_v, [pre[0] + dv])
                g1 = plsc.load_gather(m_v, [pre[1] + dv])
                g2 = plsc.load_gather(m_v, [pre[2] + dv])
                g3 = plsc.load_gather(m_v, [pre[3] + dv])
                g4 = plsc.load_gather(m_v, [pre[4] + dv])
                g5 = plsc.load_gather(m_v, [pre[5] + dv])
                g6 = plsc.load_gather(m_v, [pre[6] + dv])
                g7 = plsc.load_gather(m_v, [pre[7] + dv])
                acc = ((g0 + g1) + (g2 + g3)) + ((g4 + g5) + (g6 + g7))
                acc = jnp.maximum(acc, 0.0)
                plsc.store_scatter(out_v, [srow + dv], acc)
                return dv + 1

        pltpu.sync_copy(
            out_v, out_hbm.at[pl.ds((wid * _BPW + ch * _CH) * _D, _CH * _D)])


def _sc_call(cards_sc, m):
    mesh = plsc.VectorSubcoreMesh(core_axis_name="c", subcore_axis_name="s")
    cp = pltpu.CompilerParams()
    if "needs_layout_passes" in pltpu.CompilerParams.__dataclass_fields__:
        cp = dataclasses.replace(cp, needs_layout_passes=False)
    run = pl.kernel(
        _sc_body,
        mesh=mesh,
        compiler_params=cp,
        out_type=jax.ShapeDtypeStruct((_B * _D,), jnp.float32),
        scratch_types=[
            pltpu.VMEM((_MR * _D,), jnp.float32),
            pltpu.VMEM((_N + 1, _BPW), jnp.int32),
            pltpu.VMEM((_CH * _D,), jnp.float32),
        ],
    )
    return run(cards_sc, m)


def kernel(cards, rank_emb, suit_emb, card_emb, W, b):
    m = _build_table(rank_emb, suit_emb, card_emb, W)
    m_pad = jnp.zeros((_MR, _D), jnp.float32).at[:_C].set(m).at[_C].set(b)
    cards_t = cards.reshape(_NW, _BPW, _N).transpose(0, 2, 1)
    bias_row = jnp.full((_NW, 1, _BPW), _C, jnp.int32)
    cards_sc = jnp.concatenate([cards_t, bias_row], axis=1)
    return _sc_call(cards_sc, m_pad.reshape(-1)).reshape(_B, _D)
